# baseline (speedup 1.0000x reference)
import functools

import jax
import jax.numpy as jnp
from jax import lax
from jax.experimental import pallas as pl
from jax.experimental.pallas import tpu as pltpu
from jax.experimental.pallas import tpu_sc as plsc

_RANK = 128
_LAMBDA = 1e-4
_BATCH = 16384

_NC = 2
_NS = 16
_NW = _NC * _NS
_ROWS_PER_W = _BATCH // _NW
_CHUNK = 128
_N_CHUNKS = _ROWS_PER_W // _CHUNK


def _sc_mse_partials(u, i, r, u_feat, i_feat):
    mesh = plsc.VectorSubcoreMesh(core_axis_name="c", subcore_axis_name="s")

    @functools.partial(
        pl.kernel,
        mesh=mesh,
        compiler_params=pltpu.CompilerParams(
            needs_layout_passes=False, disable_bounds_checks=True),
        out_type=jax.ShapeDtypeStruct((_NW * 16,), jnp.float32),
        scratch_types=[
            pltpu.VMEM((_ROWS_PER_W,), jnp.int32),
            pltpu.VMEM((_ROWS_PER_W,), jnp.int32),
            pltpu.VMEM((_ROWS_PER_W + 16,), jnp.float32),
            pltpu.VMEM((2, _CHUNK, _RANK), jnp.float32),
            pltpu.VMEM((2, _CHUNK, _RANK), jnp.float32),
            pltpu.VMEM((16,), jnp.float32),
            pltpu.SemaphoreType.DMA,
            pltpu.SemaphoreType.DMA,
        ],
    )
    def sc_kernel(u_hbm, i_hbm, r_hbm, uf_hbm, if_hbm, out_hbm,
                  uidx_v, iidx_v, r_v, ubuf, ibuf, acc_v, sem0, sem1):
        wid = lax.axis_index("s") * _NC + lax.axis_index("c")
        base = wid * _ROWS_PER_W
        lane = lax.iota(jnp.int32, 16)
        sems = (sem0, sem1)

        pltpu.sync_copy(u_hbm.at[pl.ds(base, _ROWS_PER_W)], uidx_v)
        pltpu.sync_copy(i_hbm.at[pl.ds(base, _ROWS_PER_W)], iidx_v)
        pltpu.sync_copy(r_hbm.at[pl.ds(base, _ROWS_PER_W)],
                        r_v.at[pl.ds(0, _ROWS_PER_W)])

        def fire(c):
            slot = c % 2
            cu = pltpu.async_copy(
                uf_hbm.at[uidx_v.at[pl.ds(c * _CHUNK, _CHUNK)]],
                ubuf.at[slot], sems[slot])
            ci = pltpu.async_copy(
                if_hbm.at[iidx_v.at[pl.ds(c * _CHUNK, _CHUNK)]],
                ibuf.at[slot], sems[slot])
            return (cu, ci)

        pending = {0: fire(0)}
        acc = jnp.zeros((16,), jnp.float32)
        for c in range(_N_CHUNKS):
            if c + 1 < _N_CHUNKS:
                pending[c + 1] = fire(c + 1)
            cu, ci = pending.pop(c)
            cu.wait()
            ci.wait()
            slot = c % 2
            ub = ubuf.at[slot]
            ib = ibuf.at[slot]

            @plsc.parallel_loop(0, _CHUNK, step=1, unroll=4,
                                carry=jnp.float32(0.0))
            def row_body(rr, carry, ub=ub, ib=ib, c=c):
                a0 = jnp.zeros((16,), jnp.float32)
                a1 = jnp.zeros((16,), jnp.float32)
                for k in range(0, 8, 2):
                    a0 = a0 + (ub[rr, pl.ds(k * 16, 16)]
                               * ib[rr, pl.ds(k * 16, 16)])
                    a1 = a1 + (ub[rr, pl.ds((k + 1) * 16, 16)]
                               * ib[rr, pl.ds((k + 1) * 16, 16)])
                s = jnp.sum(a0 + a1)
                d = s - r_v[pl.ds(c * _CHUNK + rr, 16)][0]
                return carry + d * d

            acc = acc + jnp.where(lane == 0, row_body, 0.0)
        acc_v[...] = acc
        pltpu.sync_copy(acc_v, out_hbm.at[pl.ds(wid * 16, 16)])

    return sc_kernel(u, i, r, u_feat, i_feat)


_REG_BLK = 25000


def _tc_reg_kernel(u_ref, i_ref, out_ref):
    j = pl.program_id(0)

    @pl.when(j == 0)
    def _():
        out_ref[0, 0] = 0.0

    x = u_ref[...]
    y = i_ref[...]
    out_ref[0, 0] += jnp.sum(x * x) + jnp.sum(y * y)


def _tc_reg_loss(u_feat, i_feat):
    grid = u_feat.shape[0] // _REG_BLK
    return pl.pallas_call(
        _tc_reg_kernel,
        grid=(grid,),
        in_specs=[
            pl.BlockSpec((_REG_BLK, _RANK), lambda j: (j, 0)),
            pl.BlockSpec((_REG_BLK, _RANK), lambda j: (j, 0)),
        ],
        out_specs=pl.BlockSpec((1, 1), lambda j: (0, 0),
                               memory_space=pltpu.SMEM),
        out_shape=jax.ShapeDtypeStruct((1, 1), jnp.float32),
    )(u_feat, i_feat)


def kernel(u, i, r, u_feat, i_feat):
    mse_parts = _sc_mse_partials(u, i, r, u_feat, i_feat)
    reg = _tc_reg_loss(u_feat, i_feat)[0, 0]
    mse = jnp.sum(mse_parts) / jnp.float32(_BATCH)
    loss = mse + jnp.float32(_LAMBDA) * reg
    rmse = jnp.sqrt(mse)
    return (loss, rmse)


# speedup vs baseline: 1.0894x; 1.0894x over previous
import functools

import jax
import jax.numpy as jnp
from jax import lax
from jax.experimental import pallas as pl
from jax.experimental.pallas import tpu as pltpu
from jax.experimental.pallas import tpu_sc as plsc

_RANK = 128
_LAMBDA = 1e-4
_BATCH = 16384

_NC = 2
_NS = 16
_NW = _NC * _NS
_ROWS_PER_W = _BATCH // _NW
_CHUNK = 128
_N_CHUNKS = _ROWS_PER_W // _CHUNK


def _sc_mse_partials(u, i, r, u_feat, i_feat):
    mesh = plsc.VectorSubcoreMesh(core_axis_name="c", subcore_axis_name="s")

    @functools.partial(
        pl.kernel,
        mesh=mesh,
        compiler_params=pltpu.CompilerParams(
            needs_layout_passes=False, disable_bounds_checks=True),
        out_type=jax.ShapeDtypeStruct((_NW * 16,), jnp.float32),
        scratch_types=[
            pltpu.VMEM((_ROWS_PER_W,), jnp.int32),
            pltpu.VMEM((_ROWS_PER_W,), jnp.int32),
            pltpu.VMEM((_ROWS_PER_W + 16,), jnp.float32),
            pltpu.VMEM((2, _CHUNK, _RANK), jnp.float32),
            pltpu.VMEM((2, _CHUNK, _RANK), jnp.float32),
            pltpu.VMEM((16,), jnp.float32),
            pltpu.SemaphoreType.DMA,
            pltpu.SemaphoreType.DMA,
        ],
    )
    def sc_kernel(u_hbm, i_hbm, r_hbm, uf_hbm, if_hbm, out_hbm,
                  uidx_v, iidx_v, r_v, ubuf, ibuf, acc_v, sem0, sem1):
        wid = lax.axis_index("s") * _NC + lax.axis_index("c")
        base = wid * _ROWS_PER_W
        lane = lax.iota(jnp.int32, 16)
        sems = (sem0, sem1)

        pltpu.sync_copy(u_hbm.at[pl.ds(base, _ROWS_PER_W)], uidx_v)
        pltpu.sync_copy(i_hbm.at[pl.ds(base, _ROWS_PER_W)], iidx_v)
        pltpu.sync_copy(r_hbm.at[pl.ds(base, _ROWS_PER_W)],
                        r_v.at[pl.ds(0, _ROWS_PER_W)])

        def fire(c):
            slot = c % 2
            cu = pltpu.async_copy(
                uf_hbm.at[uidx_v.at[pl.ds(c * _CHUNK, _CHUNK)]],
                ubuf.at[slot], sems[slot])
            ci = pltpu.async_copy(
                if_hbm.at[iidx_v.at[pl.ds(c * _CHUNK, _CHUNK)]],
                ibuf.at[slot], sems[slot])
            return (cu, ci)

        pending = {0: fire(0)}
        acc = jnp.zeros((16,), jnp.float32)
        for c in range(_N_CHUNKS):
            if c + 1 < _N_CHUNKS:
                pending[c + 1] = fire(c + 1)
            cu, ci = pending.pop(c)
            cu.wait()
            ci.wait()
            slot = c % 2
            ub = ubuf.at[slot]
            ib = ibuf.at[slot]

            @plsc.parallel_loop(0, _CHUNK, step=1, unroll=4,
                                carry=jnp.float32(0.0))
            def row_body(rr, carry, ub=ub, ib=ib, c=c):
                a0 = jnp.zeros((16,), jnp.float32)
                a1 = jnp.zeros((16,), jnp.float32)
                for k in range(0, 8, 2):
                    a0 = a0 + (ub[rr, pl.ds(k * 16, 16)]
                               * ib[rr, pl.ds(k * 16, 16)])
                    a1 = a1 + (ub[rr, pl.ds((k + 1) * 16, 16)]
                               * ib[rr, pl.ds((k + 1) * 16, 16)])
                s = jnp.sum(a0 + a1)
                d = s - r_v[pl.ds(c * _CHUNK + rr, 16)][0]
                return carry + d * d

            acc = acc + jnp.where(lane == 0, row_body, 0.0)
        acc_v[...] = acc
        pltpu.sync_copy(acc_v, out_hbm.at[pl.ds(wid * 16, 16)])

    return sc_kernel(u, i, r, u_feat, i_feat)


_REG_BLK = 10000


def _tc_reg_kernel(u_ref, i_ref, out_ref):
    j = pl.program_id(0)

    @pl.when(j == 0)
    def _():
        out_ref[0, 0] = 0.0

    x = u_ref[...]
    y = i_ref[...]
    out_ref[0, 0] += jnp.sum(x * x) + jnp.sum(y * y)


def _tc_reg_loss(u_feat, i_feat):
    grid = u_feat.shape[0] // _REG_BLK
    return pl.pallas_call(
        _tc_reg_kernel,
        grid=(grid,),
        in_specs=[
            pl.BlockSpec((_REG_BLK, _RANK), lambda j: (j, 0)),
            pl.BlockSpec((_REG_BLK, _RANK), lambda j: (j, 0)),
        ],
        out_specs=pl.BlockSpec((1, 1), lambda j: (0, 0),
                               memory_space=pltpu.SMEM),
        out_shape=jax.ShapeDtypeStruct((1, 1), jnp.float32),
    )(u_feat, i_feat)


def _combine_kernel(parts_ref, reg_ref, loss_ref, rmse_ref):
    mse = jnp.sum(parts_ref[...]) * jnp.float32(1.0 / _BATCH)
    loss_ref[0, 0] = mse + jnp.float32(_LAMBDA) * reg_ref[0, 0]
    rmse_ref[0, 0] = jnp.sqrt(mse)


def _combine(mse_parts, reg):
    return pl.pallas_call(
        _combine_kernel,
        in_specs=[
            pl.BlockSpec((4, 128), lambda: (0, 0)),
            pl.BlockSpec(memory_space=pltpu.SMEM),
        ],
        out_specs=[
            pl.BlockSpec(memory_space=pltpu.SMEM),
            pl.BlockSpec(memory_space=pltpu.SMEM),
        ],
        out_shape=[
            jax.ShapeDtypeStruct((1, 1), jnp.float32),
            jax.ShapeDtypeStruct((1, 1), jnp.float32),
        ],
    )(mse_parts.reshape(4, 128), reg)


def kernel(u, i, r, u_feat, i_feat):
    mse_parts = _sc_mse_partials(u, i, r, u_feat, i_feat)
    reg = _tc_reg_loss(u_feat, i_feat)
    loss, rmse = _combine(mse_parts, reg)
    return (loss[0, 0], rmse[0, 0])
